# baseline (speedup 1.0000x reference)
import functools

import jax
import jax.numpy as jnp
from jax import lax
from jax.experimental import pallas as pl
from jax.experimental.pallas import tpu as pltpu
from jax.experimental.pallas import tpu_sc as plsc

_CHUNK = 104
_CHUNKS_PER_ROW = 2
_SPAD = _CHUNK * _CHUNKS_PER_ROW
_NBUF = 4
_UNROLL = 8
_LANES = 16


@functools.cache
def _make_sc_kernel(B, E):
    info = plsc.get_sparse_core_info()
    NC, NS = info.num_cores, info.num_subcores
    NW = NC * NS
    rows_w = B // NW
    nch = rows_w * _CHUNKS_PER_ROW
    nvec = E // _LANES
    mesh = plsc.VectorSubcoreMesh(core_axis_name="c", subcore_axis_name="s")

    @functools.partial(
        pl.kernel,
        out_type=jax.ShapeDtypeStruct((B, E), jnp.float32),
        mesh=mesh,
        compiler_params=pltpu.CompilerParams(use_tc_tiling_on_sc=False),
        scratch_types=[
            pltpu.VMEM((rows_w, _CHUNKS_PER_ROW, _CHUNK), jnp.int32),
            pltpu.VMEM((_NBUF, _CHUNK, E), jnp.float32),
            pltpu.VMEM((rows_w, E), jnp.float32),
            pltpu.SemaphoreType.DMA,
            pltpu.SemaphoreType.DMA,
            pltpu.SemaphoreType.DMA,
            pltpu.SemaphoreType.DMA,
        ],
    )
    def bow(idx_hbm, table_hbm, out_hbm, idx_v, buf_v, out_v, s0, s1, s2, s3):
        sems = (s0, s1, s2, s3)
        wid = lax.axis_index("s") * NC + lax.axis_index("c")
        base = wid * rows_w
        pltpu.sync_copy(idx_hbm.at[pl.ds(base, rows_w)], idx_v)

        def gather(row, half, slot):
            return pltpu.make_async_copy(
                table_hbm.at[idx_v.at[row, half]], buf_v.at[slot], sems[slot])

        def reduce_into(slot, acc):
            def body(jj, a):
                a = list(a)
                for u in range(_UNROLL):
                    j = jj * _UNROLL + u
                    for k in range(nvec):
                        a[k] = jnp.maximum(
                            a[k], buf_v[slot, j, pl.ds(k * _LANES, _LANES)])
                return tuple(a)
            return lax.fori_loop(0, _CHUNK // _UNROLL, body, acc)

        neg_inf = jnp.full((_LANES,), -jnp.inf, dtype=jnp.float32)

        def finalize(row, acc):
            for k in range(nvec):
                x = acc[k]
                out_v[row, pl.ds(k * _LANES, _LANES)] = (
                    1.0 - 2.0 / (jnp.exp(2.0 * x) + 1.0))

        for i in range(_NBUF):
            gather(i // _CHUNKS_PER_ROW, i % _CHUNKS_PER_ROW, i).start()

        def step(c0, last):
            rbase = c0 // _CHUNKS_PER_ROW
            acc = (neg_inf,) * nvec
            for i in range(_NBUF):
                row = rbase + i // _CHUNKS_PER_ROW
                half = i % _CHUNKS_PER_ROW
                gather(row, half, i).wait()
                acc = reduce_into(i, acc)
                if half == _CHUNKS_PER_ROW - 1:
                    finalize(row, acc)
                    acc = (neg_inf,) * nvec
                if not last:
                    gather(row + _NBUF // _CHUNKS_PER_ROW, half, i).start()

        @pl.loop(0, nch - _NBUF, step=_NBUF)
        def _(c0):
            step(c0, False)

        step(nch - _NBUF, True)
        pltpu.sync_copy(out_v, out_hbm.at[pl.ds(base, rows_w)])

    return bow


_C1 = 256


@functools.cache
def _make_transpose_kernel(V, E):
    info = plsc.get_sparse_core_info()
    NW = info.num_cores * info.num_subcores
    NC = info.num_cores
    vmain = (V // 128) * 128
    vtail = V - vmain
    nch = vmain // _C1
    nvec = E // _LANES
    mesh = plsc.VectorSubcoreMesh(core_axis_name="c", subcore_axis_name="s")

    @functools.partial(
        pl.kernel,
        out_type=jax.ShapeDtypeStruct((V * E,), jnp.float32),
        mesh=mesh,
        compiler_params=pltpu.CompilerParams(
            use_tc_tiling_on_sc=True, needs_layout_passes=False),
        scratch_types=[
            pltpu.VMEM((E, _C1), jnp.float32),
            pltpu.VMEM((_C1 * E,), jnp.float32),
        ],
    )
    def transpose(tt_hbm, stg_hbm, slab_v, out_v):
        wid = lax.axis_index("s") * NC + lax.axis_index("c")
        lanes = jax.lax.iota(jnp.int32, _LANES)

        def scatter_groups(width):
            def group(g, _):
                ibase = (lanes + g * _LANES) * E
                for e in range(E):
                    vals = slab_v[e, pl.ds(g * _LANES, _LANES)]
                    plsc.store_scatter(out_v, [ibase + e], vals)
                return 0
            lax.fori_loop(0, width // _LANES, group, 0)

        def do_chunk(v0, width):
            pltpu.sync_copy(tt_hbm.at[:, pl.ds(v0, width)],
                            slab_v.at[:, pl.ds(0, width)])
            scatter_groups(width)
            pltpu.sync_copy(out_v.at[pl.ds(0, width * E)],
                            stg_hbm.at[pl.ds(v0 * E, width * E)])

        @pl.loop(wid, nch, step=NW)
        def _(c):
            do_chunk(c * _C1, _C1)

        if vtail:
            @pl.when(wid == NW - 1)
            def _():
                for e in range(E):
                    pltpu.sync_copy(tt_hbm.at[e, pl.ds(vmain, vtail)],
                                    slab_v.at[e, pl.ds(0, vtail)])
                scatter_groups(vtail)
                pltpu.sync_copy(out_v.at[pl.ds(0, vtail * E)],
                                stg_hbm.at[pl.ds(vmain * E, vtail * E)])

    return transpose


def kernel(input, emb_table):
    B, S = input.shape
    V, E = emb_table.shape
    idx = input.astype(jnp.int32)
    idx = jnp.concatenate(
        [idx, jnp.broadcast_to(idx[:, :1], (B, _SPAD - S))], axis=1)
    idx = idx.reshape(B, _CHUNKS_PER_ROW, _CHUNK)
    staging = _make_transpose_kernel(V, E)(emb_table.T)
    table_lin = staging.reshape(V, E)
    return _make_sc_kernel(B, E)(idx, table_lin)


# speedup vs baseline: 1.0887x; 1.0887x over previous
import functools

import jax
import jax.numpy as jnp
from jax import lax
from jax.experimental import pallas as pl
from jax.experimental.pallas import tpu as pltpu
from jax.experimental.pallas import tpu_sc as plsc

_CHUNK = 128
_CHUNKS_PER_ROW = 2
_SPAD = _CHUNK * _CHUNKS_PER_ROW
_NBUF = 2
_UNROLL = 16
_LANES = 16


@functools.cache
def _make_pair_kernel(B, E):
    info = plsc.get_sparse_core_info()
    NC, NS = info.num_cores, info.num_subcores
    NW = NC * NS
    rows_w = B // NW
    nch = rows_w * _CHUNKS_PER_ROW
    nvec = E // _LANES
    W = 2 * E
    mesh = plsc.VectorSubcoreMesh(core_axis_name="c", subcore_axis_name="s")

    @functools.partial(
        pl.kernel,
        out_type=jax.ShapeDtypeStruct((B, E), jnp.float32),
        mesh=mesh,
        compiler_params=pltpu.CompilerParams(use_tc_tiling_on_sc=True,
                                             needs_layout_passes=False),
        scratch_types=[
            pltpu.VMEM((nch, _CHUNK), jnp.int32),
            pltpu.VMEM((nch, _CHUNK), jnp.int32),
            pltpu.VMEM((_NBUF, _CHUNK, W), jnp.float32),
            pltpu.VMEM((rows_w, E), jnp.float32),
            pltpu.SemaphoreType.DMA,
            pltpu.SemaphoreType.DMA,
        ],
    )
    def bow(idx_hbm, par_hbm, table_hbm, out_hbm,
            idx_v, par_v, buf_v, out_v, s0, s1):
        sems = (s0, s1)
        wid = lax.axis_index("s") * NC + lax.axis_index("c")
        base = wid * nch
        pltpu.sync_copy(idx_hbm.at[pl.ds(base, nch)], idx_v)
        pltpu.sync_copy(par_hbm.at[pl.ds(base, nch)], par_v)

        def gather(c, slot):
            return pltpu.make_async_copy(
                table_hbm.at[idx_v.at[c]], buf_v.at[slot], sems[slot])

        def reduce_into(c, slot, acc):
            def body(jj, a):
                a = list(a)
                jb = jj * _UNROLL
                p16 = par_v[c, pl.ds(jb, _LANES)]
                for u in range(_UNROLL):
                    j = jb + u
                    m = jnp.full((_LANES,), p16[u], dtype=jnp.int32) > 0
                    for k in range(nvec):
                        lo = buf_v[slot, j, pl.ds(k * _LANES, _LANES)]
                        hi = buf_v[slot, j, pl.ds(E + k * _LANES, _LANES)]
                        a[k] = jnp.maximum(a[k], jnp.where(m, hi, lo))
                return tuple(a)
            return lax.fori_loop(0, _CHUNK // _UNROLL, body, acc)

        neg_inf = jnp.full((_LANES,), -jnp.inf, dtype=jnp.float32)

        def finalize(row, acc):
            for k in range(nvec):
                x = acc[k]
                out_v[row, pl.ds(k * _LANES, _LANES)] = (
                    1.0 - 2.0 / (jnp.exp(2.0 * x) + 1.0))

        for i in range(_NBUF):
            gather(i, i).start()

        def step(c0, last):
            acc = (neg_inf,) * nvec
            for i in range(_NBUF):
                c = c0 + i
                gather(c, i).wait()
                acc = reduce_into(c, i, acc)
                if i % _CHUNKS_PER_ROW == _CHUNKS_PER_ROW - 1:
                    finalize(c // _CHUNKS_PER_ROW, acc)
                    acc = (neg_inf,) * nvec
                if not last:
                    gather(c + _NBUF, i).start()

        @pl.loop(0, nch - _NBUF, step=_NBUF)
        def _(c0):
            step(c0, False)

        step(nch - _NBUF, True)
        pltpu.sync_copy(out_v, out_hbm.at[pl.ds(wid * rows_w, rows_w)])

    return bow


def kernel(input, emb_table):
    B, S = input.shape
    V, E = emb_table.shape
    idx = input.astype(jnp.int32)
    idx = jnp.concatenate(
        [idx, jnp.broadcast_to(idx[:, :1], (B, _SPAD - S))], axis=1)
    idxhi = (idx >> 1).reshape(B * _CHUNKS_PER_ROW, _CHUNK)
    par = (idx & 1).reshape(B * _CHUNKS_PER_ROW, _CHUNK)
    table2 = emb_table.reshape(V // 2, 2 * E)
    return _make_pair_kernel(B, E)(idxhi, par, table2)


# speedup vs baseline: 2.2477x; 2.0645x over previous
import functools

import jax
import jax.numpy as jnp
from jax import lax
from jax.experimental import pallas as pl
from jax.experimental.pallas import tpu as pltpu
from jax.experimental.pallas import tpu_sc as plsc

_CHUNK = 104
_CHUNKS_PER_ROW = 2
_SPAD = _CHUNK * _CHUNKS_PER_ROW
_NBUF = 2
_UNROLL = 16
_LANES = 16


@functools.cache
def _make_row_kernel(B, E):
    info = plsc.get_sparse_core_info()
    NC, NS = info.num_cores, info.num_subcores
    NW = NC * NS
    rows_w = B // NW
    nch = rows_w * _CHUNKS_PER_ROW
    nvec = E // _LANES
    mesh = plsc.VectorSubcoreMesh(core_axis_name="c", subcore_axis_name="s")

    @functools.partial(
        pl.kernel,
        out_type=jax.ShapeDtypeStruct((B, E), jnp.float32),
        mesh=mesh,
        compiler_params=pltpu.CompilerParams(use_tc_tiling_on_sc=False),
        scratch_types=[
            pltpu.VMEM((nch, _CHUNK), jnp.int32),
            pltpu.VMEM((4, _CHUNK, E), jnp.float32),
            pltpu.VMEM((rows_w, E), jnp.float32),
            pltpu.SemaphoreType.DMA,
            pltpu.SemaphoreType.DMA,
            pltpu.SemaphoreType.DMA,
            pltpu.SemaphoreType.DMA,
        ],
    )
    def bow(idx_hbm, table_hbm, out_hbm, idx_v, buf_v, out_v, s0, s1, s2, s3):
        sems = (s0, s1, s2, s3)
        wid = lax.axis_index("s") * NC + lax.axis_index("c")
        base = wid * nch
        pltpu.sync_copy(idx_hbm.at[pl.ds(base, nch)], idx_v)

        def gather(c, slot):
            return pltpu.make_async_copy(
                table_hbm.at[idx_v.at[c]], buf_v.at[slot], sems[slot])

        def reduce_into(slot, acc):
            def body(jj, a):
                a = list(a)
                for u in range(8):
                    j = jj * 8 + u
                    for k in range(nvec):
                        a[k] = jnp.maximum(
                            a[k], buf_v[slot, j, pl.ds(k * _LANES, _LANES)])
                return tuple(a)
            return lax.fori_loop(0, _CHUNK // 8, body, acc)

        neg_inf = jnp.full((_LANES,), -jnp.inf, dtype=jnp.float32)

        def finalize(row, acc):
            for k in range(nvec):
                x = acc[k]
                out_v[row, pl.ds(k * _LANES, _LANES)] = (
                    1.0 - 2.0 / (jnp.exp(2.0 * x) + 1.0))

        for i in range(4):
            gather(i, i).start()

        def step(c0, last):
            acc = (neg_inf,) * nvec
            for i in range(4):
                c = c0 + i
                gather(c, i).wait()
                acc = reduce_into(i, acc)
                if i % _CHUNKS_PER_ROW == _CHUNKS_PER_ROW - 1:
                    finalize(c // _CHUNKS_PER_ROW, acc)
                    acc = (neg_inf,) * nvec
                if not last:
                    gather(c + 4, i).start()

        @pl.loop(0, nch - 4, step=4)
        def _(c0):
            step(c0, False)

        step(nch - 4, True)
        pltpu.sync_copy(out_v, out_hbm.at[pl.ds(wid * rows_w, rows_w)])

    return bow


def kernel(input, emb_table):
    B, S = input.shape
    V, E = emb_table.shape
    idx = input.astype(jnp.int32)
    idx = jnp.concatenate(
        [idx, jnp.broadcast_to(idx[:, :1], (B, _SPAD - S))], axis=1)
    idx2 = idx.reshape(B * _CHUNKS_PER_ROW, _CHUNK)
    return _make_row_kernel(B, E)(idx2, emb_table)
